# baseline (speedup 1.0000x reference)
import functools

import jax
import jax.numpy as jnp
from jax import lax
from jax.experimental import pallas as pl
from jax.experimental.pallas import tpu as pltpu
from jax.experimental.pallas import tpu_sc as plsc

NUM_CODES = 8192
CODE_DIM = 64
COMMITMENT_COST = 0.25
N_TOK = 16 * 576

TT = 256
TK = 1024


def _argmin_kernel(z_ref, w_ref, a_ref, c_ref, idx_ref, dmin_ref):
    k = pl.program_id(1)
    m = lax.dot_general(
        z_ref[...], w_ref[...], (((1,), (1,)), ((), ())),
        preferred_element_type=jnp.float32,
    )
    d = (a_ref[...] - 2.0 * m) + c_ref[...]
    mloc = jnp.min(d, axis=1, keepdims=True)
    iota = lax.broadcasted_iota(jnp.int32, d.shape, 1)
    iloc = jnp.min(jnp.where(d == mloc, iota, TK), axis=1, keepdims=True) + k * TK

    @pl.when(k == 0)
    def _init():
        dmin_ref[...] = mloc
        idx_ref[...] = iloc

    @pl.when(k > 0)
    def _update():
        prev = dmin_ref[...]
        better = mloc < prev
        dmin_ref[...] = jnp.where(better, mloc, prev)
        idx_ref[...] = jnp.where(better, iloc, idx_ref[...])


def _build_argmin(interpret: bool = False):
    return pl.pallas_call(
        _argmin_kernel,
        grid=(N_TOK // TT, NUM_CODES // TK),
        in_specs=[
            pl.BlockSpec((TT, CODE_DIM), lambda i, k: (i, 0)),
            pl.BlockSpec((TK, CODE_DIM), lambda i, k: (k, 0)),
            pl.BlockSpec((TT, 1), lambda i, k: (i, 0)),
            pl.BlockSpec((1, TK), lambda i, k: (0, k)),
        ],
        out_specs=[
            pl.BlockSpec((TT, 1), lambda i, k: (i, 0)),
            pl.BlockSpec((TT, 1), lambda i, k: (i, 0)),
        ],
        out_shape=[
            jax.ShapeDtypeStruct((N_TOK, 1), jnp.int32),
            jax.ShapeDtypeStruct((N_TOK, 1), jnp.float32),
        ],
        compiler_params=pltpu.CompilerParams(
            dimension_semantics=("parallel", "arbitrary"),
        ),
        interpret=interpret,
    )


_NC = 2
_NS = 16
_NW = _NC * _NS
_TPW = N_TOK // _NW
_CHUNK = 96
_NCHUNK = _TPW // _CHUNK


def _sc_gather_body(w2_hbm, idx_hbm, z_hbm, out_hbm,
                    idx_v, rows_v, z_v, out_v, sem):
    wid = lax.axis_index("s") * _NC + lax.axis_index("c")
    base = wid * _TPW
    pltpu.sync_copy(idx_hbm.at[pl.ds(base, _TPW)], idx_v)
    cps = [
        pltpu.async_copy(
            w2_hbm.at[idx_v.at[pl.ds(c * _CHUNK, _CHUNK)]],
            rows_v.at[pl.ds(c * _CHUNK, _CHUNK)],
            sem,
        )
        for c in range(_NCHUNK)
    ]
    pltpu.sync_copy(z_hbm.at[pl.ds(base, _TPW)], z_v)
    for cp in cps:
        cp.wait()

    def body(r, carry):
        for j in range(CODE_DIM // 16):
            sl = pl.ds(j * 16, 16)
            q = rows_v[r, sl]
            zz = z_v[r, sl]
            out_v[r, sl] = zz + (q - zz)
        return carry

    lax.fori_loop(0, _TPW, body, 0)
    pltpu.sync_copy(out_v, out_hbm.at[pl.ds(base, _TPW)])


@functools.lru_cache(maxsize=1)
def _build_sc_gather():
    return pl.kernel(
        _sc_gather_body,
        mesh=plsc.VectorSubcoreMesh(core_axis_name="c", subcore_axis_name="s"),
        out_type=jax.ShapeDtypeStruct((N_TOK, CODE_DIM), jnp.float32),
        scratch_types=[
            pltpu.VMEM((_TPW,), jnp.int32),
            pltpu.VMEM((_TPW, 2 * CODE_DIM), jnp.float32),
            pltpu.VMEM((_TPW, CODE_DIM), jnp.float32),
            pltpu.VMEM((_TPW, CODE_DIM), jnp.float32),
            pltpu.SemaphoreType.DMA,
        ],
    )


def kernel(z, W):
    flat_z = z.reshape(-1, CODE_DIM)
    a = jnp.sum(flat_z ** 2, axis=1, keepdims=True)
    c = jnp.sum(W ** 2, axis=1).reshape(1, NUM_CODES)
    idx2, dmin2 = _build_argmin()(flat_z, W, a, c)
    idx_flat = idx2[:, 0]
    w_wide = jnp.concatenate([W, W], axis=1)
    z_q_ste = _build_sc_gather()(w_wide, idx_flat, flat_z).reshape(z.shape)
    vq_loss = COMMITMENT_COST * (jnp.sum(dmin2[:, 0]) / (N_TOK * CODE_DIM))
    return (z_q_ste, idx_flat.reshape(z.shape[:2]), vq_loss)


# speedup vs baseline: 1.4890x; 1.4890x over previous
import functools

import jax
import jax.numpy as jnp
from jax import lax
from jax.experimental import pallas as pl
from jax.experimental.pallas import tpu as pltpu
from jax.experimental.pallas import tpu_sc as plsc

NUM_CODES = 8192
CODE_DIM = 64
COMMITMENT_COST = 0.25
N_TOK = 16 * 576

TT = 512
TK = 1024
_NKS = NUM_CODES // TK


def _argmin_kernel(w2_ref, z_ref, a_ref, c_ref, idx_ref, dmin_ref,
                   val_scr, pay_scr):
    k = pl.program_id(1)
    m2 = lax.dot_general(
        w2_ref[...], z_ref[...], (((1,), (1,)), ((), ())),
        preferred_element_type=jnp.float32,
    )
    a = a_ref[...]
    c = c_ref[...]

    @pl.when(k == 0)
    def _init():
        val_scr[...] = jnp.full((8, TT), jnp.inf, jnp.float32)
        pay_scr[...] = jnp.zeros((8, TT), jnp.float32)

    val = val_scr[...]
    pay = pay_scr[...]
    sub_iota = lax.broadcasted_iota(jnp.int32, (8, 1), 0).astype(jnp.float32)
    kbase = (k * TK).astype(jnp.float32)
    for g in range(TK // 8):
        sl = slice(g * 8, (g + 1) * 8)
        d = (a - m2[sl, :]) + c[sl, :]
        ig = sub_iota + (kbase + (g * 8))
        better = d < val
        val = jnp.where(better, d, val)
        pay = jnp.where(better, ig, pay)

    @pl.when(k < _NKS - 1)
    def _carry():
        val_scr[...] = val
        pay_scr[...] = pay

    @pl.when(k == _NKS - 1)
    def _finish():
        mcol = jnp.min(val, axis=0, keepdims=True)
        sel = jnp.where(val == mcol, pay, float(NUM_CODES))
        icol = jnp.min(sel, axis=0, keepdims=True)
        idx_ref[...] = icol.astype(jnp.int32)
        dmin_ref[...] = mcol


def _build_argmin(interpret: bool = False):
    return pl.pallas_call(
        _argmin_kernel,
        grid=(N_TOK // TT, _NKS),
        in_specs=[
            pl.BlockSpec((TK, CODE_DIM), lambda i, k: (k, 0)),
            pl.BlockSpec((TT, CODE_DIM), lambda i, k: (i, 0)),
            pl.BlockSpec((1, TT), lambda i, k: (0, i)),
            pl.BlockSpec((TK, 1), lambda i, k: (k, 0)),
        ],
        out_specs=[
            pl.BlockSpec((1, TT), lambda i, k: (0, i)),
            pl.BlockSpec((1, TT), lambda i, k: (0, i)),
        ],
        out_shape=[
            jax.ShapeDtypeStruct((1, N_TOK), jnp.int32),
            jax.ShapeDtypeStruct((1, N_TOK), jnp.float32),
        ],
        scratch_shapes=[
            pltpu.VMEM((8, TT), jnp.float32),
            pltpu.VMEM((8, TT), jnp.float32),
        ],
        compiler_params=pltpu.CompilerParams(
            dimension_semantics=("parallel", "arbitrary"),
        ),
        interpret=interpret,
    )


_NC = 2
_NS = 16
_NW = _NC * _NS
_TPW = N_TOK // _NW
_CHUNK = 96
_NCHUNK = _TPW // _CHUNK


def _sc_gather_body(w2_hbm, idx_hbm, z_hbm, out_hbm,
                    idx_v, rows_v, z_v, out_v, sem):
    wid = lax.axis_index("s") * _NC + lax.axis_index("c")
    base = wid * _TPW
    pltpu.sync_copy(idx_hbm.at[pl.ds(base, _TPW)], idx_v)
    cps = [
        pltpu.async_copy(
            w2_hbm.at[idx_v.at[pl.ds(c * _CHUNK, _CHUNK)]],
            rows_v.at[pl.ds(c * _CHUNK, _CHUNK)],
            sem,
        )
        for c in range(_NCHUNK)
    ]
    pltpu.sync_copy(z_hbm.at[pl.ds(base, _TPW)], z_v)
    for cp in cps:
        cp.wait()

    def body(r, carry):
        for j in range(CODE_DIM // 16):
            sl = pl.ds(j * 16, 16)
            q = rows_v[r, sl]
            zz = z_v[r, sl]
            out_v[r, sl] = zz + (q - zz)
        return carry

    lax.fori_loop(0, _TPW, body, 0)
    pltpu.sync_copy(out_v, out_hbm.at[pl.ds(base, _TPW)])


@functools.lru_cache(maxsize=1)
def _build_sc_gather():
    return pl.kernel(
        _sc_gather_body,
        mesh=plsc.VectorSubcoreMesh(core_axis_name="c", subcore_axis_name="s"),
        out_type=jax.ShapeDtypeStruct((N_TOK, CODE_DIM), jnp.float32),
        scratch_types=[
            pltpu.VMEM((_TPW,), jnp.int32),
            pltpu.VMEM((_TPW, 2 * CODE_DIM), jnp.float32),
            pltpu.VMEM((_TPW, CODE_DIM), jnp.float32),
            pltpu.VMEM((_TPW, CODE_DIM), jnp.float32),
            pltpu.SemaphoreType.DMA,
        ],
    )


def kernel(z, W):
    flat_z = z.reshape(-1, CODE_DIM)
    a = jnp.sum(flat_z ** 2, axis=1, keepdims=True).reshape(1, N_TOK)
    c = jnp.sum(W ** 2, axis=1).reshape(NUM_CODES, 1)
    w2 = W + W
    idx2, dmin2 = _build_argmin()(w2, flat_z, a, c)
    idx_flat = idx2[0, :]
    w_wide = jnp.concatenate([W, W], axis=1)
    z_q_ste = _build_sc_gather()(w_wide, idx_flat, flat_z).reshape(z.shape)
    vq_loss = COMMITMENT_COST * (jnp.sum(dmin2[0, :]) / (N_TOK * CODE_DIM))
    return (z_q_ste, idx_flat.reshape(z.shape[:2]), vq_loss)


# speedup vs baseline: 1.8805x; 1.2630x over previous
import functools

import jax
import jax.numpy as jnp
from jax import lax
from jax.experimental import pallas as pl
from jax.experimental.pallas import tpu as pltpu
from jax.experimental.pallas import tpu_sc as plsc

NUM_CODES = 8192
CODE_DIM = 64
COMMITMENT_COST = 0.25
N_TOK = 16 * 576

TT = 512
TK = 2048
_NKS = NUM_CODES // TK
_NACC = 4


def _argmin_kernel(w2_ref, z_ref, a_ref, c_ref, idx_ref, dmin_ref,
                   val_scr, pay_scr):
    k = pl.program_id(1)
    z = z_ref[...]
    half = TK // 2
    m2a = lax.dot_general(
        w2_ref[0:half, :], z, (((1,), (1,)), ((), ())),
        preferred_element_type=jnp.float32,
    )
    m2b = lax.dot_general(
        w2_ref[half:TK, :], z, (((1,), (1,)), ((), ())),
        preferred_element_type=jnp.float32,
    )
    a = a_ref[...]
    c = c_ref[...]

    @pl.when(k == 0)
    def _init():
        val_scr[...] = jnp.full((8 * _NACC, TT), jnp.inf, jnp.float32)
        pay_scr[...] = jnp.zeros((8 * _NACC, TT), jnp.float32)

    vals = [val_scr[8 * t:8 * (t + 1), :] for t in range(_NACC)]
    pays = [pay_scr[8 * t:8 * (t + 1), :] for t in range(_NACC)]
    kbase = (k * (TK // 8)).astype(jnp.float32)
    ncg = TK // 8
    for g in range(ncg):
        half_idx, goff = divmod(g, ncg // 2)
        src = m2a if half_idx == 0 else m2b
        sl = slice(goff * 8, (goff + 1) * 8)
        csl = slice(g * 8, (g + 1) * 8)
        d = (a - src[sl, :]) + c[csl, :]
        pg = kbase + float(g)
        t = g % _NACC
        better = d < vals[t]
        vals[t] = jnp.where(better, d, vals[t])
        pays[t] = jnp.where(better, pg, pays[t])

    @pl.when(k < _NKS - 1)
    def _carry():
        val_scr[...] = jnp.concatenate(vals, axis=0)
        pay_scr[...] = jnp.concatenate(pays, axis=0)

    @pl.when(k == _NKS - 1)
    def _finish():
        sub_iota = lax.broadcasted_iota(jnp.int32, (8, 1), 0).astype(jnp.float32)
        mval, midx = vals[0], pays[0] * 8.0 + sub_iota
        for t in range(1, _NACC):
            v2, i2 = vals[t], pays[t] * 8.0 + sub_iota
            b = (v2 < mval) | ((v2 == mval) & (i2 < midx))
            mval = jnp.where(b, v2, mval)
            midx = jnp.where(b, i2, midx)
        mcol = jnp.min(mval, axis=0, keepdims=True)
        sel = jnp.where(mval == mcol, midx, float(NUM_CODES))
        icol = jnp.min(sel, axis=0, keepdims=True)
        idx_ref[...] = icol.astype(jnp.int32)
        dmin_ref[...] = mcol


def _build_argmin(interpret: bool = False):
    return pl.pallas_call(
        _argmin_kernel,
        grid=(N_TOK // TT, _NKS),
        in_specs=[
            pl.BlockSpec((TK, CODE_DIM), lambda i, k: (k, 0)),
            pl.BlockSpec((TT, CODE_DIM), lambda i, k: (i, 0)),
            pl.BlockSpec((1, TT), lambda i, k: (0, i)),
            pl.BlockSpec((TK, 1), lambda i, k: (k, 0)),
        ],
        out_specs=[
            pl.BlockSpec((1, TT), lambda i, k: (0, i)),
            pl.BlockSpec((1, TT), lambda i, k: (0, i)),
        ],
        out_shape=[
            jax.ShapeDtypeStruct((1, N_TOK), jnp.int32),
            jax.ShapeDtypeStruct((1, N_TOK), jnp.float32),
        ],
        scratch_shapes=[
            pltpu.VMEM((8 * _NACC, TT), jnp.float32),
            pltpu.VMEM((8 * _NACC, TT), jnp.float32),
        ],
        compiler_params=pltpu.CompilerParams(
            dimension_semantics=("parallel", "arbitrary"),
        ),
        interpret=interpret,
    )


_NC = 2
_NS = 16
_NW = _NC * _NS
_TPW = N_TOK // _NW
_CHUNK = 96
_NCHUNK = _TPW // _CHUNK


def _sc_gather_body(w2_hbm, idx_hbm, z_hbm, out_hbm,
                    idx_v, rows_v, z_v, out_v, sem):
    wid = lax.axis_index("s") * _NC + lax.axis_index("c")
    base = wid * _TPW
    pltpu.sync_copy(idx_hbm.at[pl.ds(base, _TPW)], idx_v)
    cps = [
        pltpu.async_copy(
            w2_hbm.at[idx_v.at[pl.ds(c * _CHUNK, _CHUNK)]],
            rows_v.at[pl.ds(c * _CHUNK, _CHUNK)],
            sem,
        )
        for c in range(_NCHUNK)
    ]
    pltpu.sync_copy(z_hbm.at[pl.ds(base, _TPW)], z_v)
    for cp in cps:
        cp.wait()

    def body(r, carry):
        for j in range(CODE_DIM // 16):
            sl = pl.ds(j * 16, 16)
            q = rows_v[r, sl]
            zz = z_v[r, sl]
            out_v[r, sl] = zz + (q - zz)
        return carry

    lax.fori_loop(0, _TPW, body, 0)
    pltpu.sync_copy(out_v, out_hbm.at[pl.ds(base, _TPW)])


@functools.lru_cache(maxsize=1)
def _build_sc_gather():
    return pl.kernel(
        _sc_gather_body,
        mesh=plsc.VectorSubcoreMesh(core_axis_name="c", subcore_axis_name="s"),
        out_type=jax.ShapeDtypeStruct((N_TOK, CODE_DIM), jnp.float32),
        scratch_types=[
            pltpu.VMEM((_TPW,), jnp.int32),
            pltpu.VMEM((_TPW, 2 * CODE_DIM), jnp.float32),
            pltpu.VMEM((_TPW, CODE_DIM), jnp.float32),
            pltpu.VMEM((_TPW, CODE_DIM), jnp.float32),
            pltpu.SemaphoreType.DMA,
        ],
    )


def kernel(z, W):
    flat_z = z.reshape(-1, CODE_DIM)
    a = jnp.sum(flat_z ** 2, axis=1, keepdims=True).reshape(1, N_TOK)
    c = jnp.sum(W ** 2, axis=1).reshape(NUM_CODES, 1)
    w2 = W + W
    idx2, dmin2 = _build_argmin()(w2, flat_z, a, c)
    idx_flat = idx2[0, :]
    w_wide = jnp.concatenate([W, W], axis=1)
    z_q_ste = _build_sc_gather()(w_wide, idx_flat, flat_z).reshape(z.shape)
    vq_loss = COMMITMENT_COST * (jnp.sum(dmin2[0, :]) / (N_TOK * CODE_DIM))
    return (z_q_ste, idx_flat.reshape(z.shape[:2]), vq_loss)


# speedup vs baseline: 2.3367x; 1.2426x over previous
import functools

import jax
import jax.numpy as jnp
from jax import lax
from jax.experimental import pallas as pl
from jax.experimental.pallas import tpu as pltpu
from jax.experimental.pallas import tpu_sc as plsc

NUM_CODES = 8192
CODE_DIM = 64
COMMITMENT_COST = 0.25
N_TOK = 16 * 576

TT = 512
TK = 2048
_NKS = NUM_CODES // TK
_NACC = 4


_GRP = 256


def _argmin_kernel(w2_ref, z_ref, a_ref, c_ref, idx_ref, dmin_ref,
                   val_scr, pay_scr):
    k = pl.program_id(0)
    i = pl.program_id(1)
    z = z_ref[...]
    a = a_ref[...]
    c = c_ref[...]
    tds = pl.ds(i * TT, TT)

    @pl.when(k == 0)
    def _init():
        val_scr[:, tds] = jnp.full((8 * _NACC, TT), jnp.inf, jnp.float32)
        pay_scr[:, tds] = jnp.zeros((8 * _NACC, TT), jnp.float32)

    vals = [val_scr[8 * t:8 * (t + 1), tds] for t in range(_NACC)]
    pays = [pay_scr[8 * t:8 * (t + 1), tds] for t in range(_NACC)]
    kbase = (k * (TK // 8)).astype(jnp.float32)
    for j in range(TK // _GRP):
        m2j = lax.dot_general(
            w2_ref[j * _GRP:(j + 1) * _GRP, :], z, (((1,), (1,)), ((), ())),
            preferred_element_type=jnp.float32,
        )
        for gg in range(_GRP // 8):
            g = j * (_GRP // 8) + gg
            d = (a - m2j[gg * 8:(gg + 1) * 8, :]) + c[g * 8:(g + 1) * 8, :]
            pg = kbase + float(g)
            t = g % _NACC
            better = d < vals[t]
            vals[t] = jnp.where(better, d, vals[t])
            pays[t] = jnp.where(better, pg, pays[t])

    @pl.when(k < _NKS - 1)
    def _carry():
        val_scr[:, tds] = jnp.concatenate(vals, axis=0)
        pay_scr[:, tds] = jnp.concatenate(pays, axis=0)

    @pl.when(k == _NKS - 1)
    def _finish():
        sub_iota = lax.broadcasted_iota(jnp.int32, (8, 1), 0).astype(jnp.float32)
        mval, midx = vals[0], pays[0] * 8.0 + sub_iota
        for t in range(1, _NACC):
            v2, i2 = vals[t], pays[t] * 8.0 + sub_iota
            b = (v2 < mval) | ((v2 == mval) & (i2 < midx))
            mval = jnp.where(b, v2, mval)
            midx = jnp.where(b, i2, midx)
        mcol = jnp.min(mval, axis=0, keepdims=True)
        sel = jnp.where(mval == mcol, midx, float(NUM_CODES))
        icol = jnp.min(sel, axis=0, keepdims=True)
        idx_ref[...] = icol.astype(jnp.int32)
        dmin_ref[...] = mcol


def _build_argmin(interpret: bool = False):
    return pl.pallas_call(
        _argmin_kernel,
        grid=(_NKS, N_TOK // TT),
        in_specs=[
            pl.BlockSpec((TK, CODE_DIM), lambda k, i: (k, 0)),
            pl.BlockSpec((TT, CODE_DIM), lambda k, i: (i, 0)),
            pl.BlockSpec((1, TT), lambda k, i: (0, i)),
            pl.BlockSpec((TK, 1), lambda k, i: (k, 0)),
        ],
        out_specs=[
            pl.BlockSpec((1, TT), lambda k, i: (0, i)),
            pl.BlockSpec((1, TT), lambda k, i: (0, i)),
        ],
        out_shape=[
            jax.ShapeDtypeStruct((1, N_TOK), jnp.int32),
            jax.ShapeDtypeStruct((1, N_TOK), jnp.float32),
        ],
        scratch_shapes=[
            pltpu.VMEM((8 * _NACC, N_TOK), jnp.float32),
            pltpu.VMEM((8 * _NACC, N_TOK), jnp.float32),
        ],
        compiler_params=pltpu.CompilerParams(
            dimension_semantics=("arbitrary", "arbitrary"),
        ),
        interpret=interpret,
    )


_NC = 2
_NS = 16
_NW = _NC * _NS
_TPW = N_TOK // _NW
_CHUNK = 96
_NCHUNK = _TPW // _CHUNK


def _sc_gather_body(w2_hbm, idx_hbm, z_hbm, out_hbm,
                    idx_v, rows_v, z_v, out_v, sem):
    wid = lax.axis_index("s") * _NC + lax.axis_index("c")
    base = wid * _TPW
    pltpu.sync_copy(idx_hbm.at[pl.ds(base, _TPW)], idx_v)
    cps = [
        pltpu.async_copy(
            w2_hbm.at[idx_v.at[pl.ds(c * _CHUNK, _CHUNK)]],
            rows_v.at[pl.ds(c * _CHUNK, _CHUNK)],
            sem,
        )
        for c in range(_NCHUNK)
    ]
    pltpu.sync_copy(z_hbm.at[pl.ds(base, _TPW)], z_v)
    for cp in cps:
        cp.wait()

    def body(r, carry):
        for j in range(CODE_DIM // 16):
            sl = pl.ds(j * 16, 16)
            q = rows_v[r, sl]
            zz = z_v[r, sl]
            out_v[r, sl] = zz + (q - zz)
        return carry

    lax.fori_loop(0, _TPW, body, 0)
    pltpu.sync_copy(out_v, out_hbm.at[pl.ds(base, _TPW)])


@functools.lru_cache(maxsize=1)
def _build_sc_gather():
    return pl.kernel(
        _sc_gather_body,
        mesh=plsc.VectorSubcoreMesh(core_axis_name="c", subcore_axis_name="s"),
        out_type=jax.ShapeDtypeStruct((N_TOK, CODE_DIM), jnp.float32),
        scratch_types=[
            pltpu.VMEM((_TPW,), jnp.int32),
            pltpu.VMEM((_TPW, 2 * CODE_DIM), jnp.float32),
            pltpu.VMEM((_TPW, CODE_DIM), jnp.float32),
            pltpu.VMEM((_TPW, CODE_DIM), jnp.float32),
            pltpu.SemaphoreType.DMA,
        ],
    )


def kernel(z, W):
    flat_z = z.reshape(-1, CODE_DIM)
    a = jnp.sum(flat_z ** 2, axis=1, keepdims=True).reshape(1, N_TOK)
    c = jnp.sum(W ** 2, axis=1).reshape(NUM_CODES, 1)
    w2 = W + W
    idx2, dmin2 = _build_argmin()(w2, flat_z, a, c)
    idx_flat = idx2[0, :]
    w_wide = jnp.concatenate([W, W], axis=1)
    z_q_ste = _build_sc_gather()(w_wide, idx_flat, flat_z).reshape(z.shape)
    vq_loss = COMMITMENT_COST * (jnp.sum(dmin2[0, :]) / (N_TOK * CODE_DIM))
    return (z_q_ste, idx_flat.reshape(z.shape[:2]), vq_loss)
